# baseline (speedup 1.0000x reference)
import jax
import jax.numpy as jnp
from jax.experimental import pallas as pl
from jax.experimental.pallas import tpu as pltpu

_B, _L, _D = 4096, 128, 64


def _copy_body(p_ref, x_ref, o_ref):
    del p_ref
    o_ref[...] = x_ref[...]


def _permute_rows(x, p_array):
    nb = x.shape[0]
    x4 = x.reshape(nb, _L, 1, _D)
    out = pl.pallas_call(
        _copy_body,
        grid_spec=pltpu.PrefetchScalarGridSpec(
            num_scalar_prefetch=1,
            grid=(_L,),
            in_specs=[pl.BlockSpec((nb, 1, 1, _D),
                                   lambda i, p: (0, p[i], 0, 0))],
            out_specs=pl.BlockSpec((nb, 1, 1, _D),
                                   lambda i, p: (0, i, 0, 0)),
        ),
        out_shape=jax.ShapeDtypeStruct(x4.shape, jnp.float32),
    )(p_array, x4)
    return out.reshape(nb, _L, _D)


_L2 = _L // 2
_BH = _B // 2


def _pair_body(x_ref, o_ref):
    o_ref[:, :, 0, :] = x_ref[:, :, 1, :]
    o_ref[:, :, 1, :] = x_ref[:, :, 0, :]


def _rev_pairs(x):
    x4 = x.reshape(_B, _L2, 2, _D)
    out = pl.pallas_call(
        _pair_body,
        grid=(_L2, 2),
        in_specs=[pl.BlockSpec((_BH, 1, 2, _D),
                               lambda i, j: (j, _L2 - 1 - i, 0, 0))],
        out_specs=pl.BlockSpec((_BH, 1, 2, _D),
                               lambda i, j: (j, i, 0, 0)),
        out_shape=jax.ShapeDtypeStruct(x4.shape, jnp.float32),
    )(x4)
    return out.reshape(_B, _L, _D)


def kernel(inputs, p_array):
    is_rev = jnp.all(p_array == jnp.flip(jnp.arange(_L, dtype=p_array.dtype)))
    return jax.lax.cond(
        is_rev,
        lambda x, p: _rev_pairs(x),
        _permute_rows,
        inputs, p_array)


# speedup vs baseline: 1.1739x; 1.1739x over previous
import jax
import jax.numpy as jnp
from jax.experimental import pallas as pl
from jax.experimental.pallas import tpu as pltpu

_B, _L, _D = 4096, 128, 64


def _copy_body(p_ref, x_ref, o_ref):
    del p_ref
    o_ref[...] = x_ref[...]


def _permute_rows(x, p_array):
    nb = x.shape[0]
    x4 = x.reshape(nb, _L, 1, _D)
    out = pl.pallas_call(
        _copy_body,
        grid_spec=pltpu.PrefetchScalarGridSpec(
            num_scalar_prefetch=1,
            grid=(_L,),
            in_specs=[pl.BlockSpec((nb, 1, 1, _D),
                                   lambda i, p: (0, p[i], 0, 0))],
            out_specs=pl.BlockSpec((nb, 1, 1, _D),
                                   lambda i, p: (0, i, 0, 0)),
        ),
        out_shape=jax.ShapeDtypeStruct(x4.shape, jnp.float32),
    )(p_array, x4)
    return out.reshape(nb, _L, _D)


def kernel(inputs, p_array):
    return _permute_rows(inputs, p_array)


# speedup vs baseline: 1.3249x; 1.1287x over previous
import jax
import jax.numpy as jnp
from jax.experimental import pallas as pl
from jax.experimental.pallas import tpu as pltpu

_B, _L, _D = 4096, 128, 64


_NWAY = 4


def _copy_body(p_ref, *refs):
    del p_ref
    xs, o_ref = refs[:_NWAY], refs[_NWAY]
    for k, x_ref in enumerate(xs):
        o_ref[:, k] = x_ref[:, 0]


def _permute_rows(x, p_array):
    nb = x.shape[0]
    x4 = x.reshape(nb, _L, 1, _D)

    def in_map(k):
        return lambda i, p: (0, p[_NWAY * i + k], 0, 0)

    out = pl.pallas_call(
        _copy_body,
        grid_spec=pltpu.PrefetchScalarGridSpec(
            num_scalar_prefetch=1,
            grid=(_L // _NWAY,),
            in_specs=[pl.BlockSpec((nb, 1, 1, _D), in_map(k))
                      for k in range(_NWAY)],
            out_specs=pl.BlockSpec((nb, _NWAY, 1, _D),
                                   lambda i, p: (0, i, 0, 0)),
        ),
        out_shape=jax.ShapeDtypeStruct(x4.shape, jnp.float32),
    )(p_array, *([x4] * _NWAY))
    return out.reshape(nb, _L, _D)


def kernel(inputs, p_array):
    return _permute_rows(inputs, p_array)
